# Initial kernel scaffold; baseline (speedup 1.0000x reference)
import jax
import jax.numpy as jnp
from jax.experimental import pallas as pl


def kernel(feat, edge_feat, edge_index, weight, bias):
    raise NotImplementedError("write your pallas kernel here")


# speedup vs baseline: 2.7326x; 2.7326x over previous
import functools

import jax
import jax.numpy as jnp
from jax import lax
from jax.experimental import pallas as pl
from jax.experimental.pallas import tpu as pltpu
from jax.experimental.pallas import tpu_sc as plsc

N_NODES = 10000
N_EDGES = 160000
D_FEAT = 256
D_EDGE = 16
OUT_FEATS = 256

NC = 2
NS = 16
CHUNK = 128
N_CHUNKS = N_EDGES // CHUNK
HALF = D_FEAT // NC
WID = 128
N_BLOCKS = N_NODES // 16

_MESH = plsc.VectorSubcoreMesh(core_axis_name="c", subcore_axis_name="s")


def _fill_block(ref, ones_cols=()):
    zero = jnp.full((16,), 0.0, jnp.float32)
    one = jnp.full((16,), 1.0, jnp.float32)

    @pl.loop(0, 16)
    def _(i):
        for j in range(WID // 16):
            ref[i, pl.ds(j * 16, 16)] = one if j in ones_cols else zero


def _fill_rows(ref, n_rows, ones_cols=()):
    zero = jnp.full((16,), 0.0, jnp.float32)
    one = jnp.full((16,), 1.0, jnp.float32)

    @pl.loop(0, n_rows)
    def _(i):
        for j in range(WID // 16):
            ref[i, pl.ds(j * 16, 16)] = one if j in ones_cols else zero


def _zero_acc(sid, zw, acc):
    @pl.loop(sid, N_BLOCKS, step=NS)
    def _(b):
        pltpu.sync_copy(zw, acc.at[pl.ds(b * 16, 16)])


def _write_acc(sid, acc, out, row_off):
    @pl.loop(sid, N_BLOCKS, step=NS)
    def _(b):
        pltpu.sync_copy(acc.at[pl.ds(b * 16, 16)],
                        out.at[pl.ds(row_off + b * 16, 16)])


@functools.partial(
    pl.kernel,
    out_type=jax.ShapeDtypeStruct((NC * N_NODES, WID), jnp.float32),
    mesh=_MESH,
    scratch_types=[
        pltpu.VMEM((CHUNK,), jnp.int32),
        pltpu.VMEM((CHUNK,), jnp.int32),
        pltpu.VMEM((CHUNK // 8, WID), jnp.float32),
        pltpu.VMEM((CHUNK, WID), jnp.float32),
        pltpu.VMEM((CHUNK, WID), jnp.float32),
        pltpu.VMEM((16, WID), jnp.float32),
        pltpu.VMEM_SHARED((N_NODES, WID), jnp.float32),
    ],
)
def _stats_kernel(src_hbm, dst_hbm, efw_hbm, stats_out,
                  idx_s, idx_d, efw, rows_d, rows_s, zw, acc):
    cid = lax.axis_index("c")
    sid = lax.axis_index("s")
    _fill_block(zw)
    _fill_rows(rows_d, CHUNK, ones_cols=(1,))
    _fill_rows(rows_s, CHUNK, ones_cols=(2,))
    _zero_acc(sid, zw, acc)
    plsc.subcore_barrier()

    half_chunks = N_CHUNKS // NC

    @pl.loop(sid, half_chunks, step=NS)
    def _(k):
        base = cid * half_chunks + k
        off = base * CHUNK
        pltpu.sync_copy(src_hbm.at[pl.ds(off, CHUNK)], idx_s)
        pltpu.sync_copy(dst_hbm.at[pl.ds(off, CHUNK)], idx_d)
        pltpu.sync_copy(efw_hbm.at[pl.ds(base * (CHUNK // 8), CHUNK // 8)], efw)

        @pl.loop(0, CHUNK)
        def _(e):
            rows_d[e, pl.ds(0, 16)] = efw[e // 8, pl.ds((e % 8) * 16, 16)]

        pltpu.sync_copy(rows_d, acc.at[idx_d], add=True)
        pltpu.sync_copy(rows_s, acc.at[idx_s], add=True)

    plsc.subcore_barrier()
    _write_acc(sid, acc, stats_out, cid * N_NODES)


_B_ROWS = 1000


def _norm_body(feat_ref, st_ref, out_ref):
    d = st_ref[0, :, 32] + st_ref[1, :, 32]
    n = 1.0 / jnp.sqrt(jnp.maximum(d, 1.0))
    f = feat_ref[...]
    out_ref[0] = f[:, :HALF] * n[:, None]
    out_ref[1] = f[:, HALF:] * n[:, None]


def _normalize(feat, stats):
    return pl.pallas_call(
        _norm_body,
        grid=(N_NODES // _B_ROWS,),
        in_specs=[
            pl.BlockSpec((_B_ROWS, D_FEAT), lambda i: (i, 0)),
            pl.BlockSpec((NC, _B_ROWS, WID), lambda i: (0, i, 0)),
        ],
        out_specs=pl.BlockSpec((NC, _B_ROWS, HALF), lambda i: (0, i, 0)),
        out_shape=jax.ShapeDtypeStruct((NC, N_NODES, HALF), jnp.float32),
    )(feat, stats)


@functools.partial(
    pl.kernel,
    out_type=jax.ShapeDtypeStruct((NC * N_NODES, HALF), jnp.float32),
    mesh=_MESH,
    scratch_types=[
        pltpu.VMEM((CHUNK,), jnp.int32),
        pltpu.VMEM((CHUNK,), jnp.int32),
        pltpu.VMEM((CHUNK, HALF), jnp.float32),
        pltpu.VMEM((16, HALF), jnp.float32),
        pltpu.VMEM_SHARED((N_NODES, HALF), jnp.float32),
        pltpu.SemaphoreType.DMA,
    ],
)
def _agg_kernel(featn_hbm, src_hbm, dst_hbm, hfeat_out,
                idx_s, idx_d, rows, zw, acc_f, sem):
    cid = lax.axis_index("c")
    sid = lax.axis_index("s")
    _fill_block(zw)
    _zero_acc(sid, zw, acc_f)
    plsc.subcore_barrier()
    row_off = cid * N_NODES

    @pl.loop(sid, N_CHUNKS, step=NS)
    def _(k):
        off = k * CHUNK
        pltpu.sync_copy(src_hbm.at[pl.ds(off, CHUNK)], idx_s)
        pltpu.sync_copy(dst_hbm.at[pl.ds(off, CHUNK)], idx_d)
        for j in range(CHUNK // 16):
            sl = pl.ds(j * 16, 16)
            idx_s[sl] = idx_s[sl] + row_off
        pltpu.async_copy(featn_hbm.at[idx_s], rows, sem).wait()
        pltpu.sync_copy(rows, acc_f.at[idx_d], add=True)

    plsc.subcore_barrier()
    _write_acc(sid, acc_f, hfeat_out, cid * N_NODES)


def _final_body(feat_ref, hf_ref, st_ref, w_ref, b_ref, out_ref):
    acc = jnp.dot(hf_ref[0], w_ref[:HALF], preferred_element_type=jnp.float32)
    acc = acc + jnp.dot(hf_ref[1], w_ref[HALF:D_FEAT],
                        preferred_element_type=jnp.float32)
    he = st_ref[0, :, :D_EDGE] + st_ref[1, :, :D_EDGE]
    acc = acc + jnp.dot(he, w_ref[D_FEAT:], preferred_element_type=jnp.float32)
    ind = st_ref[0, :, 16] + st_ref[1, :, 16]
    nd = 1.0 / jnp.sqrt(jnp.maximum(ind, 1.0))
    rst = acc * nd[:, None] + b_ref[0]
    out_ref[:, :D_FEAT] = feat_ref[...]
    out_ref[:, D_FEAT:] = rst


def _finalize(feat, hfeat, stats, weight, bias):
    return pl.pallas_call(
        _final_body,
        grid=(N_NODES // _B_ROWS,),
        in_specs=[
            pl.BlockSpec((_B_ROWS, D_FEAT), lambda i: (i, 0)),
            pl.BlockSpec((NC, _B_ROWS, HALF), lambda i: (0, i, 0)),
            pl.BlockSpec((NC, _B_ROWS, WID), lambda i: (0, i, 0)),
            pl.BlockSpec((D_FEAT + D_EDGE, OUT_FEATS), lambda i: (0, 0)),
            pl.BlockSpec((1, OUT_FEATS), lambda i: (0, 0)),
        ],
        out_specs=pl.BlockSpec((_B_ROWS, D_FEAT + OUT_FEATS), lambda i: (i, 0)),
        out_shape=jax.ShapeDtypeStruct((N_NODES, D_FEAT + OUT_FEATS), jnp.float32),
    )(feat, hfeat, stats, weight, bias)


def kernel(feat, edge_feat, edge_index, weight, bias):
    src = edge_index[0]
    dst = edge_index[1]
    efw = edge_feat.reshape(N_EDGES // 8, WID)
    stats = _stats_kernel(src, dst, efw)
    stats = stats.reshape(NC, N_NODES, WID)
    featn = _normalize(feat, stats)
    featn_flat = featn.reshape(NC * N_NODES, HALF)
    hfeat = _agg_kernel(featn_flat, src, dst)
    hfeat = hfeat.reshape(NC, N_NODES, HALF)
    return _finalize(feat, hfeat, stats, weight, bias.reshape(1, OUT_FEATS))


# speedup vs baseline: 3.3152x; 1.2132x over previous
import functools

import jax
import jax.numpy as jnp
from jax import lax
from jax.experimental import pallas as pl
from jax.experimental.pallas import tpu as pltpu
from jax.experimental.pallas import tpu_sc as plsc

N_NODES = 10000
N_EDGES = 160000
D_FEAT = 256
D_EDGE = 16
OUT_FEATS = 256

NC = 2
NS = 16
CHUNK = 128
N_CHUNKS = N_EDGES // CHUNK
HALF = D_FEAT // NC
WID = 128
N_BLOCKS = N_NODES // 16

_MESH = plsc.VectorSubcoreMesh(core_axis_name="c", subcore_axis_name="s")


def _fill_block(ref, ones_cols=()):
    zero = jnp.full((16,), 0.0, jnp.float32)
    one = jnp.full((16,), 1.0, jnp.float32)

    @pl.loop(0, 16)
    def _(i):
        for j in range(WID // 16):
            ref[i, pl.ds(j * 16, 16)] = one if j in ones_cols else zero


def _fill_rows(ref, n_rows, ones_cols=()):
    zero = jnp.full((16,), 0.0, jnp.float32)
    one = jnp.full((16,), 1.0, jnp.float32)

    @pl.loop(0, n_rows)
    def _(i):
        for j in range(WID // 16):
            ref[i, pl.ds(j * 16, 16)] = one if j in ones_cols else zero


def _zero_acc(sid, zw, acc):
    @pl.loop(sid, N_BLOCKS, step=NS)
    def _(b):
        pltpu.sync_copy(zw, acc.at[pl.ds(b * 16, 16)])


def _write_acc(sid, acc, out, row_off):
    @pl.loop(sid, N_BLOCKS, step=NS)
    def _(b):
        pltpu.sync_copy(acc.at[pl.ds(b * 16, 16)],
                        out.at[pl.ds(row_off + b * 16, 16)])


@functools.partial(
    pl.kernel,
    out_type=jax.ShapeDtypeStruct((NC * N_NODES, WID), jnp.float32),
    mesh=_MESH,
    scratch_types=[
        pltpu.VMEM((CHUNK,), jnp.int32),
        pltpu.VMEM((CHUNK,), jnp.int32),
        pltpu.VMEM((CHUNK // 8, WID), jnp.float32),
        pltpu.VMEM((CHUNK, WID), jnp.float32),
        pltpu.VMEM((CHUNK, WID), jnp.float32),
        pltpu.VMEM((16, WID), jnp.float32),
        pltpu.VMEM_SHARED((N_NODES, WID), jnp.float32),
    ],
)
def _stats_kernel(ei_hbm, efw_hbm, stats_out,
                  idx_s, idx_d, efw, rows_d, rows_s, zw, acc):
    cid = lax.axis_index("c")
    sid = lax.axis_index("s")
    _fill_block(zw)
    _fill_rows(rows_d, CHUNK, ones_cols=(1,))
    _fill_rows(rows_s, CHUNK, ones_cols=(2,))
    _zero_acc(sid, zw, acc)
    plsc.subcore_barrier()

    half_chunks = N_CHUNKS // NC

    @pl.loop(sid, half_chunks, step=NS)
    def _(k):
        base = cid * half_chunks + k
        off = base * CHUNK
        pltpu.sync_copy(ei_hbm.at[pl.ds(off, CHUNK)], idx_s)
        pltpu.sync_copy(ei_hbm.at[pl.ds(N_EDGES + off, CHUNK)], idx_d)
        pltpu.sync_copy(efw_hbm.at[pl.ds(base * (CHUNK // 8), CHUNK // 8)], efw)

        @pl.loop(0, CHUNK)
        def _(e):
            rows_d[e, pl.ds(0, 16)] = efw[e // 8, pl.ds((e % 8) * 16, 16)]

        pltpu.sync_copy(rows_d, acc.at[idx_d], add=True)
        pltpu.sync_copy(rows_s, acc.at[idx_s], add=True)

    plsc.subcore_barrier()
    _write_acc(sid, acc, stats_out, cid * N_NODES)


_B_ROWS = 1000


def _norm_body(feat_ref, st_ref, out_ref):
    d = st_ref[0, :, 32] + st_ref[1, :, 32]
    n = 1.0 / jnp.sqrt(jnp.maximum(d, 1.0))
    f = feat_ref[...]
    out_ref[0] = f[:, :HALF] * n[:, None]
    out_ref[1] = f[:, HALF:] * n[:, None]


def _normalize(feat, stats):
    return pl.pallas_call(
        _norm_body,
        grid=(N_NODES // _B_ROWS,),
        in_specs=[
            pl.BlockSpec((_B_ROWS, D_FEAT), lambda i: (i, 0)),
            pl.BlockSpec((NC, _B_ROWS, WID), lambda i: (0, i, 0)),
        ],
        out_specs=pl.BlockSpec((NC, _B_ROWS, HALF), lambda i: (0, i, 0)),
        out_shape=jax.ShapeDtypeStruct((NC, N_NODES, HALF), jnp.float32),
    )(feat, stats)


@functools.partial(
    pl.kernel,
    out_type=jax.ShapeDtypeStruct((NC * N_NODES, HALF), jnp.float32),
    mesh=_MESH,
    scratch_types=[
        pltpu.VMEM((CHUNK,), jnp.int32),
        pltpu.VMEM((CHUNK,), jnp.int32),
        pltpu.VMEM((CHUNK,), jnp.int32),
        pltpu.VMEM((CHUNK,), jnp.int32),
        pltpu.VMEM((CHUNK, HALF), jnp.float32),
        pltpu.VMEM((CHUNK, HALF), jnp.float32),
        pltpu.VMEM((16, HALF), jnp.float32),
        pltpu.VMEM_SHARED((N_NODES, HALF), jnp.float32),
        pltpu.SemaphoreType.DMA,
        pltpu.SemaphoreType.DMA,
    ],
)
def _agg_kernel(featn_hbm, ei_hbm, hfeat_out,
                idx_s0, idx_s1, idx_d0, idx_d1, rows0, rows1, zw,
                acc_f, sem0, sem1):
    cid = lax.axis_index("c")
    sid = lax.axis_index("s")
    _fill_block(zw)
    _zero_acc(sid, zw, acc_f)
    plsc.subcore_barrier()
    row_off = cid * N_NODES
    slots = ((idx_s0, idx_d0, rows0, sem0), (idx_s1, idx_d1, rows1, sem1))

    def issue(k, slot):
        idx_s, idx_d, rows, sem = slot
        off = k * CHUNK
        pltpu.sync_copy(ei_hbm.at[pl.ds(off, CHUNK)], idx_s)
        pltpu.sync_copy(ei_hbm.at[pl.ds(N_EDGES + off, CHUNK)], idx_d)
        for j in range(CHUNK // 16):
            sl = pl.ds(j * 16, 16)
            idx_s[sl] = idx_s[sl] + row_off
        pltpu.async_copy(featn_hbm.at[idx_s], rows, sem)

    def drain(slot):
        idx_s, idx_d, rows, sem = slot
        pltpu.make_async_copy(featn_hbm.at[idx_s], rows, sem).wait()
        pltpu.sync_copy(rows, acc_f.at[idx_d], add=True)

    issue(sid, slots[0])

    @pl.loop(sid, N_CHUNKS, step=2 * NS)
    def _(k):
        @pl.when(k + NS < N_CHUNKS)
        def _():
            issue(k + NS, slots[1])

        drain(slots[0])

        @pl.when(k + 2 * NS < N_CHUNKS)
        def _():
            issue(k + 2 * NS, slots[0])

        @pl.when(k + NS < N_CHUNKS)
        def _():
            drain(slots[1])

    plsc.subcore_barrier()
    _write_acc(sid, acc_f, hfeat_out, cid * N_NODES)


def _final_body(feat_ref, hf_ref, st_ref, w_ref, b_ref, out_ref):
    acc = jnp.dot(hf_ref[0], w_ref[:HALF], preferred_element_type=jnp.float32)
    acc = acc + jnp.dot(hf_ref[1], w_ref[HALF:D_FEAT],
                        preferred_element_type=jnp.float32)
    he = st_ref[0, :, :D_EDGE] + st_ref[1, :, :D_EDGE]
    acc = acc + jnp.dot(he, w_ref[D_FEAT:], preferred_element_type=jnp.float32)
    ind = st_ref[0, :, 16] + st_ref[1, :, 16]
    nd = 1.0 / jnp.sqrt(jnp.maximum(ind, 1.0))
    rst = acc * nd[:, None] + b_ref[0]
    out_ref[:, :D_FEAT] = feat_ref[...]
    out_ref[:, D_FEAT:] = rst


def _finalize(feat, hfeat, stats, weight, bias):
    return pl.pallas_call(
        _final_body,
        grid=(N_NODES // _B_ROWS,),
        in_specs=[
            pl.BlockSpec((_B_ROWS, D_FEAT), lambda i: (i, 0)),
            pl.BlockSpec((NC, _B_ROWS, HALF), lambda i: (0, i, 0)),
            pl.BlockSpec((NC, _B_ROWS, WID), lambda i: (0, i, 0)),
            pl.BlockSpec((D_FEAT + D_EDGE, OUT_FEATS), lambda i: (0, 0)),
            pl.BlockSpec((1, OUT_FEATS), lambda i: (0, 0)),
        ],
        out_specs=pl.BlockSpec((_B_ROWS, D_FEAT + OUT_FEATS), lambda i: (i, 0)),
        out_shape=jax.ShapeDtypeStruct((N_NODES, D_FEAT + OUT_FEATS), jnp.float32),
    )(feat, hfeat, stats, weight, bias)


def kernel(feat, edge_feat, edge_index, weight, bias):
    ei = edge_index.reshape(2 * N_EDGES)
    efw = edge_feat.reshape(N_EDGES // 8, WID)
    stats = _stats_kernel(ei, efw)
    stats = stats.reshape(NC, N_NODES, WID)
    featn = _normalize(feat, stats)
    featn_flat = featn.reshape(NC * N_NODES, HALF)
    hfeat = _agg_kernel(featn_flat, ei)
    hfeat = hfeat.reshape(NC, N_NODES, HALF)
    return _finalize(feat, hfeat, stats, weight, bias.reshape(1, OUT_FEATS))


# speedup vs baseline: 3.8209x; 1.1525x over previous
import functools

import jax
import jax.numpy as jnp
from jax import lax
from jax.experimental import pallas as pl
from jax.experimental.pallas import tpu as pltpu
from jax.experimental.pallas import tpu_sc as plsc

N_NODES = 10000
N_EDGES = 160000
D_FEAT = 256
D_EDGE = 16
OUT_FEATS = 256

NC = 2
NS = 16
CHUNK = 128
N_CHUNKS = N_EDGES // CHUNK
HALF = D_FEAT // NC
WID = 128
N_BLOCKS = N_NODES // 16

_MESH = plsc.VectorSubcoreMesh(core_axis_name="c", subcore_axis_name="s")


def _fill_block(ref, ones_cols=()):
    zero = jnp.full((16,), 0.0, jnp.float32)
    one = jnp.full((16,), 1.0, jnp.float32)

    @pl.loop(0, 16)
    def _(i):
        for j in range(WID // 16):
            ref[i, pl.ds(j * 16, 16)] = one if j in ones_cols else zero


def _fill_rows(ref, n_rows, ones_cols=()):
    zero = jnp.full((16,), 0.0, jnp.float32)
    one = jnp.full((16,), 1.0, jnp.float32)

    @pl.loop(0, n_rows)
    def _(i):
        for j in range(WID // 16):
            ref[i, pl.ds(j * 16, 16)] = one if j in ones_cols else zero


def _zero_acc(sid, zw, acc):
    @pl.loop(sid, N_BLOCKS, step=NS)
    def _(b):
        pltpu.sync_copy(zw, acc.at[pl.ds(b * 16, 16)])


def _write_acc(sid, acc, out, row_off):
    @pl.loop(sid, N_BLOCKS, step=NS)
    def _(b):
        pltpu.sync_copy(acc.at[pl.ds(b * 16, 16)],
                        out.at[pl.ds(row_off + b * 16, 16)])


@functools.partial(
    pl.kernel,
    out_type=jax.ShapeDtypeStruct((NC * N_NODES, WID), jnp.float32),
    mesh=_MESH,
    scratch_types=[
        pltpu.VMEM((CHUNK,), jnp.int32),
        pltpu.VMEM((CHUNK,), jnp.int32),
        pltpu.VMEM((CHUNK,), jnp.int32),
        pltpu.VMEM((CHUNK,), jnp.int32),
        pltpu.VMEM((CHUNK // 8, WID), jnp.float32),
        pltpu.VMEM((CHUNK // 8, WID), jnp.float32),
        pltpu.VMEM((CHUNK, WID), jnp.float32),
        pltpu.VMEM((CHUNK, WID), jnp.float32),
        pltpu.VMEM((16, WID), jnp.float32),
        pltpu.VMEM_SHARED((N_NODES, WID), jnp.float32),
        pltpu.SemaphoreType.DMA,
        pltpu.SemaphoreType.DMA,
    ],
)
def _stats_kernel(ei_hbm, efw_hbm, stats_out,
                  idx_s0, idx_s1, idx_d0, idx_d1, efw0, efw1,
                  rows_d, rows_s, zw, acc, sem0, sem1):
    cid = lax.axis_index("c")
    sid = lax.axis_index("s")
    _fill_block(zw)
    _fill_rows(rows_d, CHUNK, ones_cols=(1,))
    _fill_rows(rows_s, CHUNK, ones_cols=(2,))
    _zero_acc(sid, zw, acc)
    plsc.subcore_barrier()

    half_chunks = N_CHUNKS // NC
    slots = ((idx_s0, idx_d0, efw0, sem0), (idx_s1, idx_d1, efw1, sem1))

    def loads(k, slot):
        idx_s, idx_d, efw, sem = slot
        base = cid * half_chunks + k
        off = base * CHUNK
        pltpu.async_copy(ei_hbm.at[pl.ds(off, CHUNK)], idx_s, sem)
        pltpu.async_copy(ei_hbm.at[pl.ds(N_EDGES + off, CHUNK)], idx_d, sem)
        pltpu.async_copy(
            efw_hbm.at[pl.ds(base * (CHUNK // 8), CHUNK // 8)], efw, sem)

    def waits(k, slot):
        idx_s, idx_d, efw, sem = slot
        base = cid * half_chunks + k
        off = base * CHUNK
        pltpu.make_async_copy(ei_hbm.at[pl.ds(off, CHUNK)], idx_s, sem).wait()
        pltpu.make_async_copy(
            ei_hbm.at[pl.ds(N_EDGES + off, CHUNK)], idx_d, sem).wait()
        pltpu.make_async_copy(
            efw_hbm.at[pl.ds(base * (CHUNK // 8), CHUNK // 8)], efw, sem).wait()

    def work(slot):
        idx_s, idx_d, efw, sem = slot

        @pl.loop(0, CHUNK)
        def _(e):
            rows_d[e, pl.ds(0, 16)] = efw[e // 8, pl.ds((e % 8) * 16, 16)]

        pltpu.sync_copy(rows_d, acc.at[idx_d], add=True)
        pltpu.sync_copy(rows_s, acc.at[idx_s], add=True)

    loads(sid, slots[0])

    @pl.loop(sid, half_chunks, step=2 * NS)
    def _(k):
        @pl.when(k + NS < half_chunks)
        def _():
            loads(k + NS, slots[1])

        waits(k, slots[0])
        work(slots[0])

        @pl.when(k + 2 * NS < half_chunks)
        def _():
            loads(k + 2 * NS, slots[0])

        @pl.when(k + NS < half_chunks)
        def _():
            waits(k + NS, slots[1])
            work(slots[1])

    plsc.subcore_barrier()
    _write_acc(sid, acc, stats_out, cid * N_NODES)


_B_ROWS = 1000


def _norm_body(feat_ref, st_ref, out_ref):
    d = st_ref[0, :, 32] + st_ref[1, :, 32]
    n = 1.0 / jnp.sqrt(jnp.maximum(d, 1.0))
    f = feat_ref[...]
    out_ref[0] = f[:, :HALF] * n[:, None]
    out_ref[1] = f[:, HALF:] * n[:, None]


def _normalize(feat, stats):
    return pl.pallas_call(
        _norm_body,
        grid=(N_NODES // _B_ROWS,),
        in_specs=[
            pl.BlockSpec((_B_ROWS, D_FEAT), lambda i: (i, 0)),
            pl.BlockSpec((NC, _B_ROWS, WID), lambda i: (0, i, 0)),
        ],
        out_specs=pl.BlockSpec((NC, _B_ROWS, HALF), lambda i: (0, i, 0)),
        out_shape=jax.ShapeDtypeStruct((NC, N_NODES, HALF), jnp.float32),
    )(feat, stats)


@functools.partial(
    pl.kernel,
    out_type=jax.ShapeDtypeStruct((NC * N_NODES, HALF), jnp.float32),
    mesh=_MESH,
    scratch_types=[
        pltpu.VMEM((CHUNK,), jnp.int32),
        pltpu.VMEM((CHUNK,), jnp.int32),
        pltpu.VMEM((CHUNK,), jnp.int32),
        pltpu.VMEM((CHUNK,), jnp.int32),
        pltpu.VMEM((CHUNK, HALF), jnp.float32),
        pltpu.VMEM((CHUNK, HALF), jnp.float32),
        pltpu.VMEM((16, HALF), jnp.float32),
        pltpu.VMEM_SHARED((N_NODES, HALF), jnp.float32),
        pltpu.SemaphoreType.DMA,
        pltpu.SemaphoreType.DMA,
    ],
)
def _agg_kernel(featn_hbm, ei_hbm, hfeat_out,
                idx_s0, idx_s1, idx_d0, idx_d1, rows0, rows1, zw,
                acc_f, sem0, sem1):
    cid = lax.axis_index("c")
    sid = lax.axis_index("s")
    _fill_block(zw)
    _zero_acc(sid, zw, acc_f)
    plsc.subcore_barrier()
    row_off = cid * N_NODES
    slots = ((idx_s0, idx_d0, rows0, sem0), (idx_s1, idx_d1, rows1, sem1))

    def issue(k, slot):
        idx_s, idx_d, rows, sem = slot
        off = k * CHUNK
        pltpu.sync_copy(ei_hbm.at[pl.ds(off, CHUNK)], idx_s)
        pltpu.sync_copy(ei_hbm.at[pl.ds(N_EDGES + off, CHUNK)], idx_d)
        for j in range(CHUNK // 16):
            sl = pl.ds(j * 16, 16)
            idx_s[sl] = idx_s[sl] + row_off
        pltpu.async_copy(featn_hbm.at[idx_s], rows, sem)

    def drain(slot):
        idx_s, idx_d, rows, sem = slot
        pltpu.make_async_copy(featn_hbm.at[idx_s], rows, sem).wait()
        pltpu.sync_copy(rows, acc_f.at[idx_d], add=True)

    issue(sid, slots[0])

    @pl.loop(sid, N_CHUNKS, step=2 * NS)
    def _(k):
        @pl.when(k + NS < N_CHUNKS)
        def _():
            issue(k + NS, slots[1])

        drain(slots[0])

        @pl.when(k + 2 * NS < N_CHUNKS)
        def _():
            issue(k + 2 * NS, slots[0])

        @pl.when(k + NS < N_CHUNKS)
        def _():
            drain(slots[1])

    plsc.subcore_barrier()
    _write_acc(sid, acc_f, hfeat_out, cid * N_NODES)


def _final_body(feat_ref, hf_ref, st_ref, w_ref, b_ref, out_ref):
    acc = jnp.dot(hf_ref[0], w_ref[:HALF], preferred_element_type=jnp.float32)
    acc = acc + jnp.dot(hf_ref[1], w_ref[HALF:D_FEAT],
                        preferred_element_type=jnp.float32)
    he = st_ref[0, :, :D_EDGE] + st_ref[1, :, :D_EDGE]
    acc = acc + jnp.dot(he, w_ref[D_FEAT:], preferred_element_type=jnp.float32)
    ind = st_ref[0, :, 16] + st_ref[1, :, 16]
    nd = 1.0 / jnp.sqrt(jnp.maximum(ind, 1.0))
    rst = acc * nd[:, None] + b_ref[0]
    out_ref[:, :D_FEAT] = feat_ref[...]
    out_ref[:, D_FEAT:] = rst


def _finalize(feat, hfeat, stats, weight, bias):
    return pl.pallas_call(
        _final_body,
        grid=(N_NODES // _B_ROWS,),
        in_specs=[
            pl.BlockSpec((_B_ROWS, D_FEAT), lambda i: (i, 0)),
            pl.BlockSpec((NC, _B_ROWS, HALF), lambda i: (0, i, 0)),
            pl.BlockSpec((NC, _B_ROWS, WID), lambda i: (0, i, 0)),
            pl.BlockSpec((D_FEAT + D_EDGE, OUT_FEATS), lambda i: (0, 0)),
            pl.BlockSpec((1, OUT_FEATS), lambda i: (0, 0)),
        ],
        out_specs=pl.BlockSpec((_B_ROWS, D_FEAT + OUT_FEATS), lambda i: (i, 0)),
        out_shape=jax.ShapeDtypeStruct((N_NODES, D_FEAT + OUT_FEATS), jnp.float32),
    )(feat, hfeat, stats, weight, bias)


def kernel(feat, edge_feat, edge_index, weight, bias):
    ei = edge_index.reshape(2 * N_EDGES)
    efw = edge_feat.reshape(N_EDGES // 8, WID)
    stats = _stats_kernel(ei, efw)
    stats = stats.reshape(NC, N_NODES, WID)
    featn = _normalize(feat, stats)
    featn_flat = featn.reshape(NC * N_NODES, HALF)
    hfeat = _agg_kernel(featn_flat, ei)
    hfeat = hfeat.reshape(NC, N_NODES, HALF)
    return _finalize(feat, hfeat, stats, weight, bias.reshape(1, OUT_FEATS))


# speedup vs baseline: 3.8319x; 1.0029x over previous
import functools

import jax
import jax.numpy as jnp
from jax import lax
from jax.experimental import pallas as pl
from jax.experimental.pallas import tpu as pltpu
from jax.experimental.pallas import tpu_sc as plsc

N_NODES = 10000
N_EDGES = 160000
D_FEAT = 256
D_EDGE = 16
OUT_FEATS = 256

NC = 2
NS = 16
CHUNK = 128
N_CHUNKS = N_EDGES // CHUNK
HALF = D_FEAT // NC
WID = 128
N_BLOCKS = N_NODES // 16

_MESH = plsc.VectorSubcoreMesh(core_axis_name="c", subcore_axis_name="s")


def _fill_block(ref, ones_cols=()):
    zero = jnp.full((16,), 0.0, jnp.float32)
    one = jnp.full((16,), 1.0, jnp.float32)

    @pl.loop(0, 16)
    def _(i):
        for j in range(WID // 16):
            ref[i, pl.ds(j * 16, 16)] = one if j in ones_cols else zero


def _fill_rows(ref, n_rows, ones_cols=()):
    zero = jnp.full((16,), 0.0, jnp.float32)
    one = jnp.full((16,), 1.0, jnp.float32)

    @pl.loop(0, n_rows)
    def _(i):
        for j in range(WID // 16):
            ref[i, pl.ds(j * 16, 16)] = one if j in ones_cols else zero


def _zero_acc(sid, zw, acc):
    @pl.loop(sid, N_BLOCKS, step=NS)
    def _(b):
        pltpu.sync_copy(zw, acc.at[pl.ds(b * 16, 16)])


def _write_acc(sid, acc, out, row_off):
    @pl.loop(sid, N_BLOCKS, step=NS)
    def _(b):
        pltpu.sync_copy(acc.at[pl.ds(b * 16, 16)],
                        out.at[pl.ds(row_off + b * 16, 16)])


@functools.partial(
    pl.kernel,
    out_type=jax.ShapeDtypeStruct((NC * N_NODES, WID), jnp.float32),
    mesh=_MESH,
    scratch_types=[
        pltpu.VMEM((CHUNK,), jnp.int32),
        pltpu.VMEM((CHUNK,), jnp.int32),
        pltpu.VMEM((CHUNK,), jnp.int32),
        pltpu.VMEM((CHUNK,), jnp.int32),
        pltpu.VMEM((CHUNK // 8, WID), jnp.float32),
        pltpu.VMEM((CHUNK // 8, WID), jnp.float32),
        pltpu.VMEM((CHUNK, WID), jnp.float32),
        pltpu.VMEM((CHUNK, WID), jnp.float32),
        pltpu.VMEM((16, WID), jnp.float32),
        pltpu.VMEM_SHARED((N_NODES, WID), jnp.float32),
        pltpu.SemaphoreType.DMA,
        pltpu.SemaphoreType.DMA,
    ],
)
def _stats_kernel(ei_hbm, efw_hbm, stats_out,
                  idx_s0, idx_s1, idx_d0, idx_d1, efw0, efw1,
                  rows_d, rows_s, zw, acc, sem0, sem1):
    cid = lax.axis_index("c")
    sid = lax.axis_index("s")
    _fill_block(zw)
    _fill_rows(rows_d, CHUNK, ones_cols=(1,))
    _fill_rows(rows_s, CHUNK, ones_cols=(2,))
    _zero_acc(sid, zw, acc)
    plsc.subcore_barrier()

    half_chunks = N_CHUNKS // NC
    slots = ((idx_s0, idx_d0, efw0, sem0), (idx_s1, idx_d1, efw1, sem1))

    def loads(k, slot):
        idx_s, idx_d, efw, sem = slot
        base = cid * half_chunks + k
        off = base * CHUNK
        pltpu.async_copy(ei_hbm.at[pl.ds(off, CHUNK)], idx_s, sem)
        pltpu.async_copy(ei_hbm.at[pl.ds(N_EDGES + off, CHUNK)], idx_d, sem)
        pltpu.async_copy(
            efw_hbm.at[pl.ds(base * (CHUNK // 8), CHUNK // 8)], efw, sem)

    def waits(k, slot):
        idx_s, idx_d, efw, sem = slot
        base = cid * half_chunks + k
        off = base * CHUNK
        pltpu.make_async_copy(ei_hbm.at[pl.ds(off, CHUNK)], idx_s, sem).wait()
        pltpu.make_async_copy(
            ei_hbm.at[pl.ds(N_EDGES + off, CHUNK)], idx_d, sem).wait()
        pltpu.make_async_copy(
            efw_hbm.at[pl.ds(base * (CHUNK // 8), CHUNK // 8)], efw, sem).wait()

    def work(slot):
        idx_s, idx_d, efw, sem = slot

        @pl.loop(0, CHUNK)
        def _(e):
            rows_d[e, pl.ds(0, 16)] = efw[e // 8, pl.ds((e % 8) * 16, 16)]

        pltpu.sync_copy(rows_d, acc.at[idx_d], add=True)
        pltpu.sync_copy(rows_s, acc.at[idx_s], add=True)

    loads(sid, slots[0])

    @pl.loop(sid, half_chunks, step=2 * NS)
    def _(k):
        @pl.when(k + NS < half_chunks)
        def _():
            loads(k + NS, slots[1])

        waits(k, slots[0])
        work(slots[0])

        @pl.when(k + 2 * NS < half_chunks)
        def _():
            loads(k + 2 * NS, slots[0])

        @pl.when(k + NS < half_chunks)
        def _():
            waits(k + NS, slots[1])
            work(slots[1])

    plsc.subcore_barrier()
    _write_acc(sid, acc, stats_out, cid * N_NODES)


_B_ROWS = 1000


def _norm_body(feat_ref, st_ref, out_ref):
    d = st_ref[0, :, 32] + st_ref[1, :, 32]
    n = 1.0 / jnp.sqrt(jnp.maximum(d, 1.0))
    f = feat_ref[...]
    out_ref[0] = f[:, :HALF] * n[:, None]
    out_ref[1] = f[:, HALF:] * n[:, None]


def _normalize(feat, stats):
    return pl.pallas_call(
        _norm_body,
        grid=(N_NODES // _B_ROWS,),
        in_specs=[
            pl.BlockSpec((_B_ROWS, D_FEAT), lambda i: (i, 0)),
            pl.BlockSpec((NC, _B_ROWS, WID), lambda i: (0, i, 0)),
        ],
        out_specs=pl.BlockSpec((NC, _B_ROWS, HALF), lambda i: (0, i, 0)),
        out_shape=jax.ShapeDtypeStruct((NC, N_NODES, HALF), jnp.float32),
    )(feat, stats)


@functools.partial(
    pl.kernel,
    out_type=jax.ShapeDtypeStruct((NC * N_NODES, HALF), jnp.float32),
    mesh=_MESH,
    scratch_types=[
        pltpu.VMEM((CHUNK,), jnp.int32),
        pltpu.VMEM((CHUNK,), jnp.int32),
        pltpu.VMEM((CHUNK,), jnp.int32),
        pltpu.VMEM((CHUNK,), jnp.int32),
        pltpu.VMEM((CHUNK,), jnp.int32),
        pltpu.VMEM((CHUNK,), jnp.int32),
        pltpu.VMEM((CHUNK, HALF), jnp.float32),
        pltpu.VMEM((CHUNK, HALF), jnp.float32),
        pltpu.VMEM((CHUNK, HALF), jnp.float32),
        pltpu.VMEM_SHARED((N_NODES, HALF), jnp.float32),
        pltpu.SemaphoreType.DMA,
        pltpu.SemaphoreType.DMA,
        pltpu.SemaphoreType.DMA,
    ],
)
def _agg_kernel(featn_hbm, ei3_hbm, hfeat_out,
                idx_s0, idx_s1, idx_s2, idx_d0, idx_d1, idx_d2,
                rows0, rows1, rows2, acc_f, sem0, sem1, sem2):
    cid = lax.axis_index("c")
    sid = lax.axis_index("s")
    _fill_rows(rows0, 16)
    _zero_acc(sid, rows0.at[pl.ds(0, 16)], acc_f)
    plsc.subcore_barrier()
    row_off = cid * N_NODES
    slots = ((idx_s0, idx_d0, rows0, sem0),
             (idx_s1, idx_d1, rows1, sem1),
             (idx_s2, idx_d2, rows2, sem2))

    def issue(k, slot):
        idx_s, idx_d, rows, sem = slot
        off = k * CHUNK
        pltpu.sync_copy(ei3_hbm.at[pl.ds(off, CHUNK)], idx_s)
        pltpu.sync_copy(ei3_hbm.at[pl.ds(N_EDGES + off, CHUNK)], idx_d)
        for j in range(CHUNK // 16):
            sl = pl.ds(j * 16, 16)
            idx_s[sl] = idx_s[sl] + row_off
        pltpu.async_copy(featn_hbm.at[idx_s], rows, sem)

    def drain(slot):
        idx_s, idx_d, rows, sem = slot
        pltpu.make_async_copy(featn_hbm.at[idx_s], rows, sem).wait()
        pltpu.sync_copy(rows, acc_f.at[idx_d], add=True)

    issue(sid, slots[0])

    @pl.when(sid + NS < N_CHUNKS)
    def _():
        issue(sid + NS, slots[1])

    @pl.loop(sid, N_CHUNKS, step=3 * NS)
    def _(k):
        @pl.when(k + 2 * NS < N_CHUNKS)
        def _():
            issue(k + 2 * NS, slots[2])

        drain(slots[0])

        @pl.when(k + 3 * NS < N_CHUNKS)
        def _():
            issue(k + 3 * NS, slots[0])

        @pl.when(k + NS < N_CHUNKS)
        def _():
            drain(slots[1])

        @pl.when(k + 4 * NS < N_CHUNKS)
        def _():
            issue(k + 4 * NS, slots[1])

        @pl.when(k + 2 * NS < N_CHUNKS)
        def _():
            drain(slots[2])

    plsc.subcore_barrier()
    _write_acc(sid, acc_f, hfeat_out, cid * N_NODES)


def _final_body(feat_ref, hf_ref, st_ref, w_ref, b_ref, out_ref):
    acc = jnp.dot(hf_ref[0], w_ref[:HALF], preferred_element_type=jnp.float32)
    acc = acc + jnp.dot(hf_ref[1], w_ref[HALF:D_FEAT],
                        preferred_element_type=jnp.float32)
    he = st_ref[0, :, :D_EDGE] + st_ref[1, :, :D_EDGE]
    acc = acc + jnp.dot(he, w_ref[D_FEAT:], preferred_element_type=jnp.float32)
    ind = st_ref[0, :, 16] + st_ref[1, :, 16]
    nd = 1.0 / jnp.sqrt(jnp.maximum(ind, 1.0))
    rst = acc * nd[:, None] + b_ref[0]
    out_ref[:, :D_FEAT] = feat_ref[...]
    out_ref[:, D_FEAT:] = rst


def _finalize(feat, hfeat, stats, weight, bias):
    return pl.pallas_call(
        _final_body,
        grid=(N_NODES // _B_ROWS,),
        in_specs=[
            pl.BlockSpec((_B_ROWS, D_FEAT), lambda i: (i, 0)),
            pl.BlockSpec((NC, _B_ROWS, HALF), lambda i: (0, i, 0)),
            pl.BlockSpec((NC, _B_ROWS, WID), lambda i: (0, i, 0)),
            pl.BlockSpec((D_FEAT + D_EDGE, OUT_FEATS), lambda i: (0, 0)),
            pl.BlockSpec((1, OUT_FEATS), lambda i: (0, 0)),
        ],
        out_specs=pl.BlockSpec((_B_ROWS, D_FEAT + OUT_FEATS), lambda i: (i, 0)),
        out_shape=jax.ShapeDtypeStruct((N_NODES, D_FEAT + OUT_FEATS), jnp.float32),
    )(feat, hfeat, stats, weight, bias)


def kernel(feat, edge_feat, edge_index, weight, bias):
    ei = edge_index.reshape(2 * N_EDGES)
    efw = edge_feat.reshape(N_EDGES // 8, WID)
    stats = _stats_kernel(ei, efw)
    stats = stats.reshape(NC, N_NODES, WID)
    featn = _normalize(feat, stats)
    featn_flat = featn.reshape(NC * N_NODES, HALF)
    hfeat = _agg_kernel(featn_flat, ei)
    hfeat = hfeat.reshape(NC, N_NODES, HALF)
    return _finalize(feat, hfeat, stats, weight, bias.reshape(1, OUT_FEATS))


# speedup vs baseline: 4.2707x; 1.1145x over previous
import functools

import jax
import jax.numpy as jnp
from jax import lax
from jax.experimental import pallas as pl
from jax.experimental.pallas import tpu as pltpu
from jax.experimental.pallas import tpu_sc as plsc

N_NODES = 10000
N_EDGES = 160000
D_FEAT = 256
D_EDGE = 16
OUT_FEATS = 256

NC = 2
NS = 16
CHUNK = 128
N_CHUNKS = N_EDGES // CHUNK
HALF = D_FEAT // NC
WID = 128
ZB = 80
N_BLOCKS = N_NODES // ZB

_MESH = plsc.VectorSubcoreMesh(core_axis_name="c", subcore_axis_name="s")


def _fill_rows(ref, n_rows, ones_cols=()):
    zero = jnp.full((16,), 0.0, jnp.float32)
    one = jnp.full((16,), 1.0, jnp.float32)

    @pl.loop(0, n_rows)
    def _(i):
        for j in range(WID // 16):
            ref[i, pl.ds(j * 16, 16)] = one if j in ones_cols else zero


def _zero_acc(sid, zw, acc):
    @pl.loop(sid, N_BLOCKS, step=NS)
    def _(b):
        pltpu.sync_copy(zw, acc.at[pl.ds(b * ZB, ZB)])


def _write_acc(sid, acc, out, row_off):
    @pl.loop(sid, N_BLOCKS, step=NS)
    def _(b):
        pltpu.sync_copy(acc.at[pl.ds(b * ZB, ZB)],
                        out.at[pl.ds(row_off + b * ZB, ZB)])


@functools.partial(
    pl.kernel,
    out_type=jax.ShapeDtypeStruct((NC * N_NODES, WID), jnp.float32),
    mesh=_MESH,
    scratch_types=[
        pltpu.VMEM((CHUNK,), jnp.int32),
        pltpu.VMEM((CHUNK,), jnp.int32),
        pltpu.VMEM((CHUNK,), jnp.int32),
        pltpu.VMEM((CHUNK,), jnp.int32),
        pltpu.VMEM((CHUNK // 8, WID), jnp.float32),
        pltpu.VMEM((CHUNK // 8, WID), jnp.float32),
        pltpu.VMEM((CHUNK, WID), jnp.float32),
        pltpu.VMEM((CHUNK, WID), jnp.float32),
        pltpu.VMEM((ZB, WID), jnp.float32),
        pltpu.VMEM_SHARED((N_NODES, WID), jnp.float32),
        pltpu.SemaphoreType.DMA,
        pltpu.SemaphoreType.DMA,
    ],
)
def _stats_kernel(ei_hbm, efw_hbm, stats_out,
                  idx_s0, idx_s1, idx_d0, idx_d1, efw0, efw1,
                  rows_d, rows_s, zw, acc, sem0, sem1):
    cid = lax.axis_index("c")
    sid = lax.axis_index("s")
    _fill_rows(zw, ZB)
    _fill_rows(rows_d, CHUNK, ones_cols=(1,))
    _fill_rows(rows_s, CHUNK, ones_cols=(2,))
    _zero_acc(sid, zw, acc)
    plsc.subcore_barrier()

    half_chunks = N_CHUNKS // NC
    slots = ((idx_s0, idx_d0, efw0, sem0), (idx_s1, idx_d1, efw1, sem1))

    def loads(k, slot):
        idx_s, idx_d, efw, sem = slot
        base = cid * half_chunks + k
        off = base * CHUNK
        pltpu.async_copy(ei_hbm.at[pl.ds(off, CHUNK)], idx_s, sem)
        pltpu.async_copy(ei_hbm.at[pl.ds(N_EDGES + off, CHUNK)], idx_d, sem)
        pltpu.async_copy(
            efw_hbm.at[pl.ds(base * (CHUNK // 8), CHUNK // 8)], efw, sem)

    def waits(k, slot):
        idx_s, idx_d, efw, sem = slot
        base = cid * half_chunks + k
        off = base * CHUNK
        pltpu.make_async_copy(ei_hbm.at[pl.ds(off, CHUNK)], idx_s, sem).wait()
        pltpu.make_async_copy(
            ei_hbm.at[pl.ds(N_EDGES + off, CHUNK)], idx_d, sem).wait()
        pltpu.make_async_copy(
            efw_hbm.at[pl.ds(base * (CHUNK // 8), CHUNK // 8)], efw, sem).wait()

    def work(slot):
        idx_s, idx_d, efw, sem = slot

        @pl.loop(0, CHUNK)
        def _(e):
            rows_d[e, pl.ds(0, 16)] = efw[e // 8, pl.ds((e % 8) * 16, 16)]

        pltpu.sync_copy(rows_d, acc.at[idx_d], add=True)
        pltpu.sync_copy(rows_s, acc.at[idx_s], add=True)

    loads(sid, slots[0])

    @pl.loop(sid, half_chunks, step=2 * NS)
    def _(k):
        @pl.when(k + NS < half_chunks)
        def _():
            loads(k + NS, slots[1])

        waits(k, slots[0])
        work(slots[0])

        @pl.when(k + 2 * NS < half_chunks)
        def _():
            loads(k + 2 * NS, slots[0])

        @pl.when(k + NS < half_chunks)
        def _():
            waits(k + NS, slots[1])
            work(slots[1])

    plsc.subcore_barrier()
    _write_acc(sid, acc, stats_out, cid * N_NODES)


_B_ROWS = 1000


def _norm_body(feat_ref, st_ref, out_ref):
    d = st_ref[0, :, 32] + st_ref[1, :, 32]
    n = 1.0 / jnp.sqrt(jnp.maximum(d, 1.0))
    f = feat_ref[...]
    out_ref[0] = f[:, :HALF] * n[:, None]
    out_ref[1] = f[:, HALF:] * n[:, None]


def _normalize(feat, stats):
    return pl.pallas_call(
        _norm_body,
        grid=(N_NODES // _B_ROWS,),
        in_specs=[
            pl.BlockSpec((_B_ROWS, D_FEAT), lambda i: (i, 0)),
            pl.BlockSpec((NC, _B_ROWS, WID), lambda i: (0, i, 0)),
        ],
        out_specs=pl.BlockSpec((NC, _B_ROWS, HALF), lambda i: (0, i, 0)),
        out_shape=jax.ShapeDtypeStruct((NC, N_NODES, HALF), jnp.float32),
    )(feat, stats)


@functools.partial(
    pl.kernel,
    out_type=jax.ShapeDtypeStruct((NC * N_NODES, HALF), jnp.float32),
    mesh=_MESH,
    scratch_types=[
        pltpu.VMEM((CHUNK,), jnp.int32),
        pltpu.VMEM((CHUNK,), jnp.int32),
        pltpu.VMEM((CHUNK,), jnp.int32),
        pltpu.VMEM((CHUNK,), jnp.int32),
        pltpu.VMEM((CHUNK,), jnp.int32),
        pltpu.VMEM((CHUNK,), jnp.int32),
        pltpu.VMEM((CHUNK, HALF), jnp.float32),
        pltpu.VMEM((CHUNK, HALF), jnp.float32),
        pltpu.VMEM((CHUNK, HALF), jnp.float32),
        pltpu.VMEM_SHARED((N_NODES, HALF), jnp.float32),
        pltpu.SemaphoreType.DMA,
        pltpu.SemaphoreType.DMA,
        pltpu.SemaphoreType.DMA,
    ],
)
def _agg_kernel(featn_hbm, ei3_hbm, hfeat_out,
                idx_s0, idx_s1, idx_s2, idx_d0, idx_d1, idx_d2,
                rows0, rows1, rows2, acc_f, sem0, sem1, sem2):
    cid = lax.axis_index("c")
    sid = lax.axis_index("s")
    _fill_rows(rows0, ZB)
    _zero_acc(sid, rows0.at[pl.ds(0, ZB)], acc_f)
    plsc.subcore_barrier()
    row_off = cid * N_NODES
    slots = ((idx_s0, idx_d0, rows0, sem0),
             (idx_s1, idx_d1, rows1, sem1),
             (idx_s2, idx_d2, rows2, sem2))

    def issue(k, slot):
        idx_s, idx_d, rows, sem = slot
        off = k * CHUNK
        pltpu.sync_copy(ei3_hbm.at[pl.ds(off, CHUNK)], idx_s)
        pltpu.sync_copy(ei3_hbm.at[pl.ds(N_EDGES + off, CHUNK)], idx_d)
        for j in range(CHUNK // 16):
            sl = pl.ds(j * 16, 16)
            idx_s[sl] = idx_s[sl] + row_off
        pltpu.async_copy(featn_hbm.at[idx_s], rows, sem)

    def drain(slot):
        idx_s, idx_d, rows, sem = slot
        pltpu.make_async_copy(featn_hbm.at[idx_s], rows, sem).wait()
        pltpu.sync_copy(rows, acc_f.at[idx_d], add=True)

    issue(sid, slots[0])

    @pl.when(sid + NS < N_CHUNKS)
    def _():
        issue(sid + NS, slots[1])

    @pl.loop(sid, N_CHUNKS, step=3 * NS)
    def _(k):
        @pl.when(k + 2 * NS < N_CHUNKS)
        def _():
            issue(k + 2 * NS, slots[2])

        drain(slots[0])

        @pl.when(k + 3 * NS < N_CHUNKS)
        def _():
            issue(k + 3 * NS, slots[0])

        @pl.when(k + NS < N_CHUNKS)
        def _():
            drain(slots[1])

        @pl.when(k + 4 * NS < N_CHUNKS)
        def _():
            issue(k + 4 * NS, slots[1])

        @pl.when(k + 2 * NS < N_CHUNKS)
        def _():
            drain(slots[2])

    plsc.subcore_barrier()
    _write_acc(sid, acc_f, hfeat_out, cid * N_NODES)


def _final_body(feat_ref, hf_ref, st_ref, w_ref, b_ref, out_ref):
    acc = jnp.dot(hf_ref[0], w_ref[:HALF], preferred_element_type=jnp.float32)
    acc = acc + jnp.dot(hf_ref[1], w_ref[HALF:D_FEAT],
                        preferred_element_type=jnp.float32)
    he = st_ref[0, :, :D_EDGE] + st_ref[1, :, :D_EDGE]
    acc = acc + jnp.dot(he, w_ref[D_FEAT:], preferred_element_type=jnp.float32)
    ind = st_ref[0, :, 16] + st_ref[1, :, 16]
    nd = 1.0 / jnp.sqrt(jnp.maximum(ind, 1.0))
    rst = acc * nd[:, None] + b_ref[0]
    out_ref[:, :D_FEAT] = feat_ref[...]
    out_ref[:, D_FEAT:] = rst


def _finalize(feat, hfeat, stats, weight, bias):
    return pl.pallas_call(
        _final_body,
        grid=(N_NODES // _B_ROWS,),
        in_specs=[
            pl.BlockSpec((_B_ROWS, D_FEAT), lambda i: (i, 0)),
            pl.BlockSpec((NC, _B_ROWS, HALF), lambda i: (0, i, 0)),
            pl.BlockSpec((NC, _B_ROWS, WID), lambda i: (0, i, 0)),
            pl.BlockSpec((D_FEAT + D_EDGE, OUT_FEATS), lambda i: (0, 0)),
            pl.BlockSpec((1, OUT_FEATS), lambda i: (0, 0)),
        ],
        out_specs=pl.BlockSpec((_B_ROWS, D_FEAT + OUT_FEATS), lambda i: (i, 0)),
        out_shape=jax.ShapeDtypeStruct((N_NODES, D_FEAT + OUT_FEATS), jnp.float32),
    )(feat, hfeat, stats, weight, bias)


def kernel(feat, edge_feat, edge_index, weight, bias):
    ei = edge_index.reshape(2 * N_EDGES)
    efw = edge_feat.reshape(N_EDGES // 8, WID)
    stats = _stats_kernel(ei, efw)
    stats = stats.reshape(NC, N_NODES, WID)
    featn = _normalize(feat, stats)
    featn_flat = featn.reshape(NC * N_NODES, HALF)
    hfeat = _agg_kernel(featn_flat, ei)
    hfeat = hfeat.reshape(NC, N_NODES, HALF)
    return _finalize(feat, hfeat, stats, weight, bias.reshape(1, OUT_FEATS))
